# baseline (speedup 1.0000x reference)
import functools

import jax
import jax.numpy as jnp
from jax import lax
from jax.experimental import pallas as pl
from jax.experimental.pallas import tpu as pltpu
from jax.experimental.pallas import tpu_sc as plsc

_ROWS = 2
_NBUF = 2


@functools.cache
def _make_gather(batch, seq, V, D, n_workers, nc):
    assert batch % (n_workers * _ROWS * _NBUF) == 0
    chunk = _ROWS * seq
    rows_per_w = batch // n_workers
    n_chunks = rows_per_w // _ROWS
    n_groups = n_chunks // _NBUF
    mesh = plsc.VectorSubcoreMesh(core_axis_name="c", subcore_axis_name="s")

    scratch = (
        [pltpu.VMEM((chunk,), jnp.int32) for _ in range(_NBUF)]
        + [pltpu.VMEM((chunk, D), jnp.float32) for _ in range(_NBUF)]
        + [pltpu.SemaphoreType.DMA] * (3 * _NBUF)
    )

    @functools.partial(
        pl.kernel,
        mesh=mesh,
        out_type=jax.ShapeDtypeStruct((batch, seq, D), jnp.float32),
        scratch_types=scratch,
        compiler_params=pltpu.CompilerParams(use_tc_tiling_on_sc=False),
    )
    def gather_kernel(idx_hbm, table_hbm, out_hbm, *bufs):
        idx_v = bufs[0:_NBUF]
        rows_v = bufs[_NBUF : 2 * _NBUF]
        s_i = bufs[2 * _NBUF : 3 * _NBUF]
        s_g = bufs[3 * _NBUF : 4 * _NBUF]
        s_o = bufs[4 * _NBUF : 5 * _NBUF]

        wid = lax.axis_index("s") * nc + lax.axis_index("c")
        base = wid * rows_per_w

        def write_out(b, row0, wait):
            for r in range(_ROWS):
                cp = pltpu.make_async_copy(
                    rows_v[b].at[pl.ds(r * seq, seq)],
                    out_hbm.at[row0 + r],
                    s_o[b],
                )
                cp.wait() if wait else cp.start()

        for b in range(_NBUF):
            pltpu.async_copy(
                idx_hbm.at[pl.ds((base + b * _ROWS) * seq, chunk)], idx_v[b], s_i[b]
            )
        for b in range(_NBUF):
            pltpu.make_async_copy(
                idx_hbm.at[pl.ds((base + b * _ROWS) * seq, chunk)], idx_v[b], s_i[b]
            ).wait()
            pltpu.async_copy(table_hbm.at[idx_v[b]], rows_v[b], s_g[b])

        def body(t, carry):
            j0 = t * _NBUF
            for b in range(_NBUF):
                row0 = base + (j0 + b) * _ROWS
                pltpu.make_async_copy(
                    table_hbm.at[idx_v[b]], rows_v[b], s_g[b]
                ).wait()
                write_out(b, row0, wait=False)
                pltpu.async_copy(
                    idx_hbm.at[pl.ds((row0 + _NBUF * _ROWS) * seq, chunk)],
                    idx_v[b],
                    s_i[b],
                )
            for b in range(_NBUF):
                row0 = base + (j0 + b) * _ROWS
                pltpu.make_async_copy(
                    idx_hbm.at[pl.ds((row0 + _NBUF * _ROWS) * seq, chunk)],
                    idx_v[b],
                    s_i[b],
                ).wait()
                write_out(b, row0, wait=True)
                pltpu.async_copy(table_hbm.at[idx_v[b]], rows_v[b], s_g[b])
            return carry

        lax.fori_loop(0, n_groups - 1, body, 0)

        last = base + (n_chunks - _NBUF) * _ROWS
        for b in range(_NBUF):
            row0 = last + b * _ROWS
            pltpu.make_async_copy(table_hbm.at[idx_v[b]], rows_v[b], s_g[b]).wait()
            write_out(b, row0, wait=False)
        for b in range(_NBUF):
            row0 = last + b * _ROWS
            write_out(b, row0, wait=True)

    return gather_kernel


def kernel(indices, table):
    batch, seq = indices.shape
    vocab, dim = table.shape
    info = plsc.get_sparse_core_info()
    n_workers = info.num_cores * info.num_subcores
    flat = indices.reshape(-1)
    return _make_gather(batch, seq, vocab, dim, n_workers, info.num_cores)(
        flat, table
    )
